# baseline (speedup 1.0000x reference)
import jax
import jax.numpy as jnp
from jax import lax
from jax.experimental import pallas as pl
from jax.experimental.pallas import tpu as pltpu
from jax.experimental.pallas import tpu_sc as plsc

_SIZE_IN = 17
_B = 1024
_L = 16
_NS = 16
_IDX_PAD = 144
_CHUNKS = _B // _L


def _scale_row_out(row_v, out_v, out_hbm, slot):
    def _chunk(j, carry):
        val = row_v[0, pl.ds(j * _L, _L)]
        out_v[pl.ds(j * _L, _L)] = val * 12.0 - 6.0
        return carry

    lax.fori_loop(0, _CHUNKS, _chunk, 0, unroll=4)
    pltpu.sync_copy(out_v, out_hbm.at[pl.ds(slot * _B, _B)])


def _body(tbl_hbm, out_hbm, idx_v, row_v, out_v, sem):
    wid = lax.axis_index("s")
    iota = lax.iota(jnp.int32, _L)
    for i in range(_IDX_PAD // _L):
        lo = 1 << (2 * i)
        hi = (1 << (2 * i + 1)) if 2 * i + 1 < _SIZE_IN else 0
        idx_v[pl.ds(i * _L, _L)] = jnp.where(
            iota == 0, lo, jnp.where(iota == 8, hi, 0)
        )
    pltpu.async_copy(
        tbl_hbm.at[idx_v.at[pl.ds(wid * 8, 1)]], row_v, sem
    ).wait()
    _scale_row_out(row_v, out_v, out_hbm, wid)

    @pl.when(wid == 0)
    def _():
        pltpu.async_copy(
            tbl_hbm.at[idx_v.at[pl.ds(8 * (_SIZE_IN - 1), 1)]], row_v, sem
        ).wait()
        _scale_row_out(row_v, out_v, out_hbm, _SIZE_IN - 1)


def kernel(input_var):
    tbl = input_var.T
    mesh = plsc.VectorSubcoreMesh(
        core_axis_name="c", subcore_axis_name="s", num_cores=1
    )
    out_flat = pl.kernel(
        _body,
        out_type=jax.ShapeDtypeStruct((_SIZE_IN * _B,), jnp.float32),
        mesh=mesh,
        compiler_params=pltpu.CompilerParams(needs_layout_passes=False),
        scratch_types=[
            pltpu.VMEM((_IDX_PAD,), jnp.int32),
            pltpu.VMEM((1, _B), jnp.float32),
            pltpu.VMEM((_B,), jnp.float32),
            pltpu.SemaphoreType.DMA,
        ],
    )(tbl)
    return out_flat.reshape(_SIZE_IN, _B).T


# speedup vs baseline: 1.0446x; 1.0446x over previous
import jax
import jax.numpy as jnp
from jax import lax
from jax.experimental import pallas as pl
from jax.experimental.pallas import tpu as pltpu
from jax.experimental.pallas import tpu_sc as plsc

_SIZE_IN = 17
_B = 1024
_L = 16
_NS = 16
_IDX_PAD = 144
_CHUNKS = _B // _L


def _scale_row_out(row_v, out_v, out_hbm, slot):
    for j in range(_CHUNKS):
        val = row_v[0, pl.ds(j * _L, _L)]
        out_v[pl.ds(j * _L, _L)] = val * 12.0 - 6.0
    pltpu.sync_copy(out_v, out_hbm.at[pl.ds(slot * _B, _B)])


def _body(tbl_hbm, out_hbm, idx_v, row_v, out_v, sem):
    wid = lax.axis_index("s")
    iota = lax.iota(jnp.int32, _L)
    for i in range(_IDX_PAD // _L):
        lo = 1 << (2 * i)
        hi = (1 << (2 * i + 1)) if 2 * i + 1 < _SIZE_IN else 0
        idx_v[pl.ds(i * _L, _L)] = jnp.where(
            iota == 0, lo, jnp.where(iota == 8, hi, 0)
        )
    pltpu.async_copy(
        tbl_hbm.at[idx_v.at[pl.ds(wid * 8, 1)]], row_v, sem
    ).wait()
    _scale_row_out(row_v, out_v, out_hbm, wid)

    @pl.when(wid == 0)
    def _():
        pltpu.async_copy(
            tbl_hbm.at[idx_v.at[pl.ds(8 * (_SIZE_IN - 1), 1)]], row_v, sem
        ).wait()
        _scale_row_out(row_v, out_v, out_hbm, _SIZE_IN - 1)


def kernel(input_var):
    tbl = input_var.T
    mesh = plsc.VectorSubcoreMesh(
        core_axis_name="c", subcore_axis_name="s", num_cores=1
    )
    out_flat = pl.kernel(
        _body,
        out_type=jax.ShapeDtypeStruct((_SIZE_IN * _B,), jnp.float32),
        mesh=mesh,
        compiler_params=pltpu.CompilerParams(needs_layout_passes=False),
        scratch_types=[
            pltpu.VMEM((_IDX_PAD,), jnp.int32),
            pltpu.VMEM((1, _B), jnp.float32),
            pltpu.VMEM((_B,), jnp.float32),
            pltpu.SemaphoreType.DMA,
        ],
    )(tbl)
    return out_flat.reshape(_SIZE_IN, _B).T


# speedup vs baseline: 1.0571x; 1.0120x over previous
import jax
import jax.numpy as jnp
from jax import lax
from jax.experimental import pallas as pl
from jax.experimental.pallas import tpu as pltpu
from jax.experimental.pallas import tpu_sc as plsc

_SIZE_IN = 17
_B = 1024
_L = 16
_IDX_PAD = 144
_CHUNKS = _B // _L


def _scale_row_out(row_v, out_v, out_hbm, slot):
    for j in range(_CHUNKS):
        val = row_v[0, pl.ds(j * _L, _L)]
        out_v[pl.ds(j * _L, _L)] = val * 12.0 - 6.0
    pltpu.sync_copy(out_v, out_hbm.at[pl.ds(slot * _B, _B)])


def _body(tbl_hbm, out_hbm, idx_v, row_v, out_v, sem):
    wid = lax.axis_index("s")
    iota = lax.iota(jnp.int32, _L)
    for i in range(_IDX_PAD // _L):
        lo = 1 << (2 * i)
        hi = (1 << (2 * i + 1)) if 2 * i + 1 < _SIZE_IN else 0
        idx_v[pl.ds(i * _L, _L)] = jnp.where(
            iota == 0, lo, jnp.where(iota == 8, hi, 0)
        )
    pltpu.async_copy(
        tbl_hbm.at[idx_v.at[pl.ds(wid * 8, 1)]], row_v, sem
    ).wait()
    _scale_row_out(row_v, out_v, out_hbm, wid)

    @pl.when(wid == 0)
    def _():
        pltpu.async_copy(
            tbl_hbm.at[idx_v.at[pl.ds(8 * (_SIZE_IN - 1), 1)]], row_v, sem
        ).wait()
        _scale_row_out(row_v, out_v, out_hbm, _SIZE_IN - 1)


def kernel(input_var):
    tbl = input_var.T
    mesh = plsc.VectorSubcoreMesh(
        core_axis_name="c", subcore_axis_name="s", num_cores=1
    )
    out_flat = pl.kernel(
        _body,
        out_type=jax.ShapeDtypeStruct((_SIZE_IN * _B,), jnp.float32),
        mesh=mesh,
        compiler_params=pltpu.CompilerParams(needs_layout_passes=False),
        scratch_types=[
            pltpu.VMEM((_IDX_PAD,), jnp.int32),
            pltpu.VMEM((1, _B), jnp.float32),
            pltpu.VMEM((_B,), jnp.float32),
            pltpu.SemaphoreType.DMA,
        ],
    )(tbl)
    return out_flat.reshape(_SIZE_IN, _B).T
